# baseline (speedup 1.0000x reference)
import functools

import jax
import jax.numpy as jnp
from jax import lax
from jax.experimental import pallas as pl
from jax.experimental.pallas import tpu as pltpu
from jax.experimental.pallas import tpu_sc as plsc

_L = 16
_NC = 2
_NS = 16
_NW = _NC * _NS
_C = 80
_D = 5


def _precompute_tables(z, W1, b1):
    n, d = z.shape
    h = W1.shape[0]

    def body(z_ref, w1_ref, b1_ref, za_ref, zb_ref):
        zz = z_ref[...]
        w1 = w1_ref[...]
        za = lax.dot_general(zz, w1[:, :d], (((1,), (1,)), ((), ())),
                             preferred_element_type=jnp.float32)
        zb = lax.dot_general(zz, w1[:, d:], (((1,), (1,)), ((), ())),
                             preferred_element_type=jnp.float32)
        za_ref[...] = (za + b1_ref[...]).astype(jnp.bfloat16)
        zb_ref[...] = zb.astype(jnp.bfloat16)

    return pl.pallas_call(
        body,
        out_shape=(jax.ShapeDtypeStruct((n, h), jnp.bfloat16),
                   jax.ShapeDtypeStruct((n, h), jnp.bfloat16)),
    )(z, W1, b1.reshape(1, h))


@functools.lru_cache(maxsize=None)
def _make_sc_scorer(n_edges, hid):
    epw = n_edges // _NW
    nch = epw // _C
    npair = hid // 2

    mesh = plsc.VectorSubcoreMesh(core_axis_name="c", subcore_axis_name="s")

    @functools.partial(
        pl.kernel,
        out_type=jax.ShapeDtypeStruct((_NW, nch, _C), jnp.float32),
        mesh=mesh,
        compiler_params=pltpu.CompilerParams(
            needs_layout_passes=False, use_tc_tiling_on_sc=False),
        scratch_types=[
            pltpu.VMEM((nch, _C), jnp.int32),
            pltpu.VMEM((nch, _C), jnp.int32),
            pltpu.VMEM((_D * _C, npair), jnp.int32),
            pltpu.VMEM((_D * _C, npair), jnp.int32),
            pltpu.VMEM((nch, _C), jnp.float32),
            pltpu.VMEM((npair + 1, _L), jnp.int32),
        ] + [pltpu.SemaphoreType.DMA] * (2 * _D),
    )
    def scorer(eidx_hbm, za_hbm, zb_hbm, wv_hbm, out_hbm,
               src_v, dst_v, rows_a, rows_b, out_v, wv_v,
               *sems):
        wid = lax.axis_index("s") * _NC + lax.axis_index("c")
        pltpu.sync_copy(eidx_hbm.at[0, wid], src_v)
        pltpu.sync_copy(eidx_hbm.at[1, wid], dst_v)
        pltpu.sync_copy(wv_hbm, wv_v)

        sems_a = sems[:_D]
        sems_b = sems[_D:]

        def gather_start(g, slot):
            pltpu.async_copy(za_hbm.at[src_v.at[g]],
                             rows_a.at[pl.ds(slot * _C, _C)], sems_a[slot])
            pltpu.async_copy(zb_hbm.at[dst_v.at[g]],
                             rows_b.at[pl.ds(slot * _C, _C)], sems_b[slot])

        def gather_wait(g, slot):
            pltpu.make_async_copy(
                za_hbm.at[src_v.at[g]],
                rows_a.at[pl.ds(slot * _C, _C)], sems_a[slot]).wait()
            pltpu.make_async_copy(
                zb_hbm.at[dst_v.at[g]],
                rows_b.at[pl.ds(slot * _C, _C)], sems_b[slot]).wait()

        b2v = plsc.bitcast(wv_v[npair], jnp.float32)
        zero = jnp.zeros((_L,), jnp.float32)
        one = jnp.ones((_L,), jnp.float32)
        zero_b = jnp.zeros((2 * _L,), jnp.bfloat16)
        lane = lax.iota(jnp.int32, _L)

        def compute(g, slot):
            for blk in range(_C // _L):
                rv = lane + (slot * _C + blk * _L)
                acc = [zero, zero, zero, zero]
                for j in range(npair):
                    wj = jnp.full((_L,), j, jnp.int32)
                    ga = plsc.load_gather(rows_a, [rv, wj])
                    gb = plsc.load_gather(rows_b, [rv, wj])
                    t = jnp.maximum(
                        plsc.bitcast(ga, jnp.bfloat16)
                        + plsc.bitcast(gb, jnp.bfloat16), zero_b)
                    te, to = plsc.unpack(
                        t * plsc.bitcast(wv_v[j], jnp.bfloat16),
                        format=plsc.PackFormat.INTERLEAVED,
                        preferred_element_type=jnp.float32)
                    k = 2 * (j % 2)
                    acc[k] = acc[k] + te
                    acc[k + 1] = acc[k + 1] + to
                x = (acc[0] + acc[1]) + (acc[2] + acc[3]) + b2v
                out_v[g, pl.ds(blk * _L, _L)] = one / (one + jnp.exp(-x))

        for s in range(_D - 1):
            gather_start(s, s)

        def ring_body(i, carry):
            for j in range(_D):
                g = _D * i + j
                gather_start(g + _D - 1, (j + _D - 1) % _D)
                gather_wait(g, j)
                compute(g, j)
            return carry

        lax.fori_loop(0, nch // _D - 1, ring_body, 0)
        base = nch - _D
        for j in range(_D):
            g = base + j
            if j < 1:
                gather_start(g + _D - 1, (j + _D - 1) % _D)
            gather_wait(g, j)
            compute(g, j)

        pltpu.sync_copy(out_v, out_hbm.at[wid])

    return scorer


def kernel(z, edge_index, W1, b1, W2, b2):
    n_edges = edge_index.shape[1]
    hid = W1.shape[0]
    n = z.shape[0]
    za, zb = _precompute_tables(z, W1, b1)
    za_p = lax.bitcast_convert_type(za.reshape(n, hid // 2, 2), jnp.int32)
    zb_p = lax.bitcast_convert_type(zb.reshape(n, hid // 2, 2), jnp.int32)
    eidx = edge_index.astype(jnp.int32).reshape(2, _NW, n_edges // (_NW * _C), _C)
    w2p = lax.bitcast_convert_type(
        W2.reshape(hid // 2, 2).astype(jnp.bfloat16), jnp.int32)
    wv = jnp.concatenate([
        jnp.broadcast_to(w2p[:, None], (hid // 2, _L)),
        lax.bitcast_convert_type(
            jnp.full((1, _L), b2[0], jnp.float32), jnp.int32),
    ]).astype(jnp.int32)
    out = _make_sc_scorer(n_edges, hid)(eidx, za_p, zb_p, wv)
    return out.reshape(-1)


# speedup vs baseline: 1.6619x; 1.6619x over previous
import functools

import jax
import jax.numpy as jnp
from jax import lax
from jax.experimental import pallas as pl
from jax.experimental.pallas import tpu as pltpu
from jax.experimental.pallas import tpu_sc as plsc

_L = 16
_NC = 2
_NS = 16
_NW = _NC * _NS
_C = 80
_D = 5


def _precompute_tables(z, W1, b1, wpos, wneg):
    n, d = z.shape
    h = W1.shape[0]

    def body(z_ref, w1_ref, b1_ref, wp_ref, wn_ref, za_ref, zb_ref):
        zz = z_ref[...]
        w1 = w1_ref[...]
        za = lax.dot_general(zz, w1[:, :d], (((1,), (1,)), ((), ())),
                             preferred_element_type=jnp.float32) + b1_ref[...]
        zb = lax.dot_general(zz, w1[:, d:], (((1,), (1,)), ((), ())),
                             preferred_element_type=jnp.float32)
        wp = wp_ref[...]
        wn = wn_ref[...]
        za_ref[...] = jnp.concatenate([za * wp, za * wn], 1).astype(jnp.bfloat16)
        zb_ref[...] = jnp.concatenate([zb * wp, zb * wn], 1).astype(jnp.bfloat16)

    return pl.pallas_call(
        body,
        out_shape=(jax.ShapeDtypeStruct((n, 2 * h), jnp.bfloat16),
                   jax.ShapeDtypeStruct((n, 2 * h), jnp.bfloat16)),
    )(z, W1, b1.reshape(1, h), wpos, wneg)


@functools.lru_cache(maxsize=None)
def _make_sc_scorer(n_edges, hid):
    epw = n_edges // _NW
    nch = epw // _C
    npair = hid // 2
    nword = 2 * npair

    mesh = plsc.VectorSubcoreMesh(core_axis_name="c", subcore_axis_name="s")

    @functools.partial(
        pl.kernel,
        out_type=jax.ShapeDtypeStruct((_NW, nch, _C), jnp.float32),
        mesh=mesh,
        compiler_params=pltpu.CompilerParams(
            needs_layout_passes=False, use_tc_tiling_on_sc=False),
        scratch_types=[
            pltpu.VMEM((nch, _C), jnp.int32),
            pltpu.VMEM((nch, _C), jnp.int32),
            pltpu.VMEM((_D * _C, nword), jnp.int32),
            pltpu.VMEM((_D * _C, nword), jnp.int32),
            pltpu.VMEM((nch, _C), jnp.float32),
            pltpu.VMEM((1, _L), jnp.float32),
        ] + [pltpu.SemaphoreType.DMA] * (2 * _D),
    )
    def scorer(eidx_hbm, za_hbm, zb_hbm, b2_hbm, out_hbm,
               src_v, dst_v, rows_a, rows_b, out_v, b2_v,
               *sems):
        wid = lax.axis_index("s") * _NC + lax.axis_index("c")
        pltpu.sync_copy(eidx_hbm.at[0, wid], src_v)
        pltpu.sync_copy(eidx_hbm.at[1, wid], dst_v)
        pltpu.sync_copy(b2_hbm, b2_v)

        sems_a = sems[:_D]
        sems_b = sems[_D:]

        def gather_start(g, slot):
            pltpu.async_copy(za_hbm.at[src_v.at[g]],
                             rows_a.at[pl.ds(slot * _C, _C)], sems_a[slot])
            pltpu.async_copy(zb_hbm.at[dst_v.at[g]],
                             rows_b.at[pl.ds(slot * _C, _C)], sems_b[slot])

        def gather_wait(g, slot):
            pltpu.make_async_copy(
                za_hbm.at[src_v.at[g]],
                rows_a.at[pl.ds(slot * _C, _C)], sems_a[slot]).wait()
            pltpu.make_async_copy(
                zb_hbm.at[dst_v.at[g]],
                rows_b.at[pl.ds(slot * _C, _C)], sems_b[slot]).wait()

        b2v = b2_v[0]
        zero = jnp.zeros((_L,), jnp.float32)
        one = jnp.ones((_L,), jnp.float32)
        zero_b = jnp.zeros((2 * _L,), jnp.bfloat16)
        lane = lax.iota(jnp.int32, _L)

        _U = 8

        def compute(g, slot):
            for blk in range(_C // _L):
                rv = lane + (slot * _C + blk * _L)

                def jbody(jj, acc, rv=rv):
                    acc = list(acc)
                    for u in range(_U):
                        j = jj * _U + u
                        rotp = (lane + j) & (npair - 1)
                        rotn = rotp | npair
                        gap = plsc.load_gather(rows_a, [rv, rotp])
                        gbp = plsc.load_gather(rows_b, [rv, rotp])
                        gan = plsc.load_gather(rows_a, [rv, rotn])
                        gbn = plsc.load_gather(rows_b, [rv, rotn])
                        tp = jnp.maximum(
                            plsc.bitcast(gap, jnp.bfloat16)
                            + plsc.bitcast(gbp, jnp.bfloat16), zero_b)
                        tn = jnp.minimum(
                            plsc.bitcast(gan, jnp.bfloat16)
                            + plsc.bitcast(gbn, jnp.bfloat16), zero_b)
                        te, to = plsc.unpack(
                            tp + tn, format=plsc.PackFormat.INTERLEAVED,
                            preferred_element_type=jnp.float32)
                        k = 2 * (u % 2)
                        acc[k] = acc[k] + te
                        acc[k + 1] = acc[k + 1] + to
                    return tuple(acc)

                acc = lax.fori_loop(0, npair // _U, jbody,
                                    (zero, zero, zero, zero))
                x = (acc[0] + acc[1]) + (acc[2] + acc[3]) + b2v
                out_v[g, pl.ds(blk * _L, _L)] = one / (one + jnp.exp(-x))

        for s in range(_D - 1):
            gather_start(s, s)

        def ring_body(i, carry):
            for j in range(_D):
                g = _D * i + j
                nxt = g + _D - 1

                @pl.when(nxt < nch)
                def _():
                    gather_start(nxt, (j + _D - 1) % _D)

                gather_wait(g, j)
                compute(g, j)
            return carry

        lax.fori_loop(0, nch // _D, ring_body, 0)

        pltpu.sync_copy(out_v, out_hbm.at[wid])

    return scorer


def kernel(z, edge_index, W1, b1, W2, b2):
    n_edges = edge_index.shape[1]
    hid = W1.shape[0]
    n = z.shape[0]
    w2f = W2.reshape(1, hid).astype(jnp.float32)
    za, zb = _precompute_tables(z, W1, b1,
                                jnp.maximum(w2f, 0.0), jnp.minimum(w2f, 0.0))
    za_p = lax.bitcast_convert_type(za.reshape(n, hid, 2), jnp.int32)
    zb_p = lax.bitcast_convert_type(zb.reshape(n, hid, 2), jnp.int32)
    eidx = edge_index.astype(jnp.int32).reshape(2, _NW, n_edges // (_NW * _C), _C)
    b2s = jnp.full((1, _L), b2[0], jnp.float32)
    out = _make_sc_scorer(n_edges, hid)(eidx, za_p, zb_p, b2s)
    return out.reshape(-1)


# speedup vs baseline: 2.4698x; 1.4862x over previous
import functools

import jax
import jax.numpy as jnp
from jax import lax
from jax.experimental import pallas as pl
from jax.experimental.pallas import tpu as pltpu
from jax.experimental.pallas import tpu_sc as plsc

_L = 16
_NC = 2
_NS = 16
_NW = _NC * _NS
_C = 80
_D = 5


def _precompute_tables(z, W1, b1, wabs):
    n, d = z.shape
    h = W1.shape[0]

    def body(z_ref, w1_ref, b1_ref, wa_ref, za_ref, zb_ref):
        zz = z_ref[...]
        w1 = w1_ref[...]
        za = lax.dot_general(zz, w1[:, :d], (((1,), (1,)), ((), ())),
                             preferred_element_type=jnp.float32) + b1_ref[...]
        zb = lax.dot_general(zz, w1[:, d:], (((1,), (1,)), ((), ())),
                             preferred_element_type=jnp.float32)
        wa = wa_ref[...]
        za_ref[...] = (za * wa).astype(jnp.bfloat16)
        zb_ref[...] = (zb * wa).astype(jnp.bfloat16)

    return pl.pallas_call(
        body,
        out_shape=(jax.ShapeDtypeStruct((n, h), jnp.bfloat16),
                   jax.ShapeDtypeStruct((n, h), jnp.bfloat16)),
    )(z, W1, b1.reshape(1, h), wabs)


@functools.lru_cache(maxsize=None)
def _make_sc_scorer(n_edges, hid):
    epw = n_edges // _NW
    nch = epw // _C
    npair = hid // 2

    mesh = plsc.VectorSubcoreMesh(core_axis_name="c", subcore_axis_name="s")

    @functools.partial(
        pl.kernel,
        out_type=jax.ShapeDtypeStruct((_NW, nch, _C), jnp.float32),
        mesh=mesh,
        compiler_params=pltpu.CompilerParams(
            needs_layout_passes=False, use_tc_tiling_on_sc=False),
        scratch_types=[
            pltpu.VMEM((nch, _C), jnp.int32),
            pltpu.VMEM((nch, _C), jnp.int32),
            pltpu.VMEM((_D * _C, npair), jnp.int32),
            pltpu.VMEM((_D * _C, npair), jnp.int32),
            pltpu.VMEM((nch, _C), jnp.float32),
            pltpu.VMEM((1, npair), jnp.int32),
            pltpu.VMEM((1, _L), jnp.float32),
        ] + [pltpu.SemaphoreType.DMA] * (2 * _D),
    )
    def scorer(eidx_hbm, za_hbm, zb_hbm, sg_hbm, b2_hbm, out_hbm,
               src_v, dst_v, rows_a, rows_b, out_v, sg_v, b2_v,
               *sems):
        wid = lax.axis_index("s") * _NC + lax.axis_index("c")
        pltpu.sync_copy(eidx_hbm.at[0, wid], src_v)
        pltpu.sync_copy(eidx_hbm.at[1, wid], dst_v)
        pltpu.sync_copy(sg_hbm, sg_v)
        pltpu.sync_copy(b2_hbm, b2_v)

        sems_a = sems[:_D]
        sems_b = sems[_D:]

        def gather_start(g, slot):
            pltpu.async_copy(za_hbm.at[src_v.at[g]],
                             rows_a.at[pl.ds(slot * _C, _C)], sems_a[slot])
            pltpu.async_copy(zb_hbm.at[dst_v.at[g]],
                             rows_b.at[pl.ds(slot * _C, _C)], sems_b[slot])

        def gather_wait(g, slot):
            pltpu.make_async_copy(
                za_hbm.at[src_v.at[g]],
                rows_a.at[pl.ds(slot * _C, _C)], sems_a[slot]).wait()
            pltpu.make_async_copy(
                zb_hbm.at[dst_v.at[g]],
                rows_b.at[pl.ds(slot * _C, _C)], sems_b[slot]).wait()

        b2v = b2_v[0]
        zero = jnp.zeros((_L,), jnp.float32)
        one = jnp.ones((_L,), jnp.float32)
        zero_b = jnp.zeros((2 * _L,), jnp.bfloat16)
        lane = lax.iota(jnp.int32, _L)

        _U = 16
        zrow = jnp.zeros((_L,), jnp.int32)

        def compute(g, slot):
            for blk in range(_C // _L):
                rv = lane + (slot * _C + blk * _L)

                def jbody(jj, acc, rv=rv):
                    acc = list(acc)
                    for u in range(_U):
                        j = jj * _U + u
                        rot = (lane + j) & (npair - 1)
                        ga = plsc.load_gather(rows_a, [rv, rot])
                        gb = plsc.load_gather(rows_b, [rv, rot])
                        gs = plsc.load_gather(sg_v, [zrow, rot])
                        t = jnp.maximum(
                            plsc.bitcast(ga, jnp.bfloat16)
                            + plsc.bitcast(gb, jnp.bfloat16), zero_b)
                        te, to = plsc.unpack(
                            t * plsc.bitcast(gs, jnp.bfloat16),
                            format=plsc.PackFormat.INTERLEAVED,
                            preferred_element_type=jnp.float32)
                        k = 2 * (u % 2)
                        acc[k] = acc[k] + te
                        acc[k + 1] = acc[k + 1] + to
                    return tuple(acc)

                acc = lax.fori_loop(0, npair // _U, jbody,
                                    (zero, zero, zero, zero))
                x = (acc[0] + acc[1]) + (acc[2] + acc[3]) + b2v
                out_v[g, pl.ds(blk * _L, _L)] = one / (one + jnp.exp(-x))

        for s in range(_D - 1):
            gather_start(s, s)

        def ring_body(i, carry):
            for j in range(_D):
                g = _D * i + j
                nxt = g + _D - 1

                @pl.when(nxt < nch)
                def _():
                    gather_start(nxt, (j + _D - 1) % _D)

                gather_wait(g, j)
                compute(g, j)
            return carry

        lax.fori_loop(0, nch // _D, ring_body, 0)

        pltpu.sync_copy(out_v, out_hbm.at[wid])

    return scorer


def kernel(z, edge_index, W1, b1, W2, b2):
    n_edges = edge_index.shape[1]
    hid = W1.shape[0]
    n = z.shape[0]
    w2f = W2.reshape(1, hid).astype(jnp.float32)
    za, zb = _precompute_tables(z, W1, b1, jnp.abs(w2f))
    za_p = lax.bitcast_convert_type(za.reshape(n, hid // 2, 2), jnp.int32)
    zb_p = lax.bitcast_convert_type(zb.reshape(n, hid // 2, 2), jnp.int32)
    eidx = edge_index.astype(jnp.int32).reshape(2, _NW, n_edges // (_NW * _C), _C)
    sgn = lax.bitcast_convert_type(
        jnp.sign(w2f).reshape(hid // 2, 2).astype(jnp.bfloat16),
        jnp.int32).reshape(1, hid // 2)
    b2s = jnp.full((1, _L), b2[0], jnp.float32)
    out = _make_sc_scorer(n_edges, hid)(eidx, za_p, zb_p, sgn, b2s)
    return out.reshape(-1)
